# baseline (speedup 1.0000x reference)
import functools

import jax
import jax.numpy as jnp
from jax import lax
from jax.experimental import pallas as pl
from jax.experimental.pallas import tpu as pltpu
from jax.experimental.pallas import tpu_sc as plsc

ALPHA = 0.2
MOMENTUM = 0.8
BQ = 512
TOK_CHUNK = 256


def _rownorm(x):
    n = jnp.sqrt(jnp.sum(x * x, axis=1, keepdims=True))
    return x / jnp.clip(n, 1e-12)


def _img_stats(step, cache, tok_ref, sc_ref, as_ref, cm_ref, cnt_ref, m):
    tok = tok_ref[...]
    n = jnp.sqrt(jnp.sum(tok * tok, axis=1, keepdims=True))
    invn = 1.0 / jnp.clip(n, 1e-12)
    b = tok * invn
    s = jax.lax.dot_general(b, cache, (((1,), (1,)), ((), ())))
    rv = jnp.max(s, axis=1, keepdims=True)
    jidx = jax.lax.broadcasted_iota(jnp.int32, s.shape, 1)
    amin = jnp.min(jnp.where(s == rv, jidx, m), axis=1, keepdims=True)
    sc_ref[...] = jnp.exp(rv) * invn
    as_ref[...] = amin
    pcm = jnp.max(s, axis=0, keepdims=True)
    pc = jnp.sum((jidx == amin).astype(jnp.float32), axis=0, keepdims=True)

    @pl.when(step == 0)
    def _():
        cm_ref[...] = pcm
        cnt_ref[...] = pc

    @pl.when(step != 0)
    def _():
        cm_ref[...] = jnp.maximum(cm_ref[...], pcm)
        cnt_ref[...] = cnt_ref[...] + pc


def _stats_kernel(i4_ref, i8_ref, i12_ref, cache_ref,
                  sc4_ref, as4_ref, cm4_ref, cnt4_ref,
                  sc8_ref, as8_ref, cm8_ref, cnt8_ref,
                  sc12_ref, as12_ref, cm12_ref, cnt12_ref, *, m):
    step = pl.program_id(0)
    cache = cache_ref[...]
    _img_stats(step, cache, i4_ref, sc4_ref, as4_ref, cm4_ref, cnt4_ref, m)
    _img_stats(step, cache, i8_ref, sc8_ref, as8_ref, cm8_ref, cnt8_ref, m)
    _img_stats(step, cache, i12_ref, sc12_ref, as12_ref, cm12_ref, cnt12_ref, m)


def _read_kernel(text_ref, cache_ref, w1_ref, w2_ref, tf_ref, loss_ref):
    step = pl.program_id(0)
    cache = cache_ref[...]
    text = text_ref[...]
    base = _rownorm(text)
    s = jax.lax.dot_general(base, cache, (((1,), (1,)), ((), ())))
    p = jnp.exp(s - jnp.max(s, axis=1, keepdims=True))
    p = p / jnp.sum(p, axis=1, keepdims=True)
    fine = jax.lax.dot_general(p, cache, (((1,), (0,)), ((), ())))
    tf = ALPHA * (jax.lax.dot_general(text, w1_ref[...], (((1,), (1,)), ((), ())))
                  + jax.lax.dot_general(fine, w2_ref[...], (((1,), (1,)), ((), ())))) + text
    tf_ref[...] = tf
    ab = jnp.abs(_rownorm(tf) - text)
    pa = jnp.sum(jnp.sum(ab, axis=1, keepdims=True), axis=0, keepdims=True)

    @pl.when(step == 0)
    def _():
        loss_ref[...] = pa

    @pl.when(step != 0)
    def _():
        loss_ref[...] = loss_ref[...] + pa


def _sc_body(t4_ref, t8_ref, t12_ref, sc4_ref, as4_ref, sc8_ref, as8_ref,
             sc12_ref, as12_ref, o4_ref, o8_ref, o12_ref,
             tok_v, scale_v, as_v, sums_v, *, c, m, nc):
    cid = lax.axis_index("c")
    sid = lax.axis_index("s")
    wid = sid * nc + cid
    lane0 = wid * 16
    lanes = lax.iota(jnp.int32, 16)

    for tok_hbm, scl_hbm, asn_hbm, out_hbm in (
            (t4_ref, sc4_ref, as4_ref, o4_ref),
            (t8_ref, sc8_ref, as8_ref, o8_ref),
            (t12_ref, sc12_ref, as12_ref, o12_ref)):
        pltpu.sync_copy(scl_hbm, scale_v)
        pltpu.sync_copy(asn_hbm, as_v)

        def zero_body(i, _):
            sums_v[i, :] = jnp.zeros((16,), jnp.float32)
            return 0
        lax.fori_loop(0, m, zero_body, 0)

        def chunk_body(ci, _):
            base = ci * TOK_CHUNK
            pltpu.sync_copy(
                tok_hbm.at[pl.ds(base, TOK_CHUNK), pl.ds(lane0, 16)], tok_v)

            def tok_body(r, _):
                iv = jnp.zeros((16,), jnp.int32) + (base + r)
                scv = plsc.load_gather(scale_v, [iv])
                av = plsc.load_gather(as_v, [iv])
                row = tok_v[r, :]
                plsc.addupdate_scatter(sums_v, [av, lanes], row * scv)
                return 0
            lax.fori_loop(0, TOK_CHUNK, tok_body, 0)
            return 0
        lax.fori_loop(0, c // TOK_CHUNK, chunk_body, 0)

        pltpu.sync_copy(sums_v, out_hbm.at[:, pl.ds(lane0, 16)])


def _update_kernel(cache_ref, s4_ref, c4_ref, m4_ref, s8_ref, c8_ref, m8_ref,
                   s12_ref, c12_ref, m12_ref, out_ref):
    cache = cache_ref[...]
    acc = None
    for s_ref, c_ref, m_ref in ((s4_ref, c4_ref, m4_ref),
                                (s8_ref, c8_ref, m8_ref),
                                (s12_ref, c12_ref, m12_ref)):
        sums = s_ref[...] * jnp.exp(-m_ref[...])
        upd = jnp.where(c_ref[...] > 0.0,
                        MOMENTUM * cache + (1.0 - MOMENTUM) * sums,
                        cache)
        u = _rownorm(upd)
        acc = u if acc is None else acc + u
    out_ref[...] = acc / 3.0


def kernel(text_token, image_token4, image_token8, image_token12, cache, W):
    c, d = text_token.shape
    m = cache.shape[0]
    nblk = c // BQ
    w1 = W[:, :d]
    w2 = W[:, d:]
    f32 = jnp.float32

    tok_spec = pl.BlockSpec((BQ, d), lambda i: (i, 0))
    full2 = lambda shape: pl.BlockSpec(shape, lambda i: (0, 0))
    col_spec = pl.BlockSpec((BQ, 1), lambda i: (i, 0))

    stats_out_shape = tuple(
        x for _ in range(3) for x in (
            jax.ShapeDtypeStruct((c, 1), f32),
            jax.ShapeDtypeStruct((c, 1), jnp.int32),
            jax.ShapeDtypeStruct((1, m), f32),
            jax.ShapeDtypeStruct((1, m), f32),
        )
    )
    stats_out_spec = tuple(
        x for _ in range(3) for x in (col_spec, col_spec, full2((1, m)),
                                      full2((1, m)))
    )
    (sc4, as4, cm4, cnt4, sc8, as8, cm8, cnt8,
     sc12, as12, cm12, cnt12) = pl.pallas_call(
        functools.partial(_stats_kernel, m=m),
        grid=(nblk,),
        in_specs=[tok_spec, tok_spec, tok_spec, full2((m, d))],
        out_specs=stats_out_spec,
        out_shape=stats_out_shape,
    )(image_token4, image_token8, image_token12, cache)

    info = plsc.get_sparse_core_info()
    nc = info.num_cores
    mesh = plsc.VectorSubcoreMesh(core_axis_name="c", subcore_axis_name="s")
    sc_fn = pl.kernel(
        functools.partial(_sc_body, c=c, m=m, nc=nc),
        mesh=mesh,
        compiler_params=pltpu.CompilerParams(use_tc_tiling_on_sc=False,
                                             needs_layout_passes=False),
        out_type=tuple(jax.ShapeDtypeStruct((m, d), f32) for _ in range(3)),
        scratch_types=[
            pltpu.VMEM((TOK_CHUNK, 16), f32),
            pltpu.VMEM((c,), f32),
            pltpu.VMEM((c,), jnp.int32),
            pltpu.VMEM((m, 16), f32),
        ],
    )
    s4, s8, s12 = sc_fn(
        image_token4, image_token8, image_token12,
        sc4.reshape(c), as4.reshape(c),
        sc8.reshape(c), as8.reshape(c),
        sc12.reshape(c), as12.reshape(c))

    text_fine, loss_sum = pl.pallas_call(
        _read_kernel,
        grid=(nblk,),
        in_specs=[tok_spec, full2((m, d)), full2((d, d)), full2((d, d))],
        out_specs=(tok_spec, full2((1, 1))),
        out_shape=(jax.ShapeDtypeStruct((c, d), f32),
                   jax.ShapeDtypeStruct((1, 1), f32)),
    )(text_token, cache, w1, w2)

    bm = 688 if m % 688 == 0 else m
    row_spec = pl.BlockSpec((bm, d), lambda i: (i, 0))
    cnt_spec = pl.BlockSpec((bm, 1), lambda i: (i, 0))
    updated_cache = pl.pallas_call(
        _update_kernel,
        grid=(m // bm,),
        in_specs=[row_spec] + [x for _ in range(3)
                               for x in (row_spec, cnt_spec, cnt_spec)],
        out_specs=row_spec,
        out_shape=jax.ShapeDtypeStruct((m, d), f32),
    )(cache, s4, cnt4.T, cm4.T, s8, cnt8.T, cm8.T, s12, cnt12.T, cm12.T)

    loss = loss_sum[0, 0] / (c * d)
    return (text_fine, loss, updated_cache)


# speedup vs baseline: 1.4334x; 1.4334x over previous
import functools

import jax
import jax.numpy as jnp
from jax import lax
from jax.experimental import pallas as pl
from jax.experimental.pallas import tpu as pltpu
from jax.experimental.pallas import tpu_sc as plsc

ALPHA = 0.2
MOMENTUM = 0.8
BQ = 512
TOK_CHUNK = 256


def _rownorm(x):
    n = jnp.sqrt(jnp.sum(x * x, axis=1, keepdims=True))
    return x / jnp.clip(n, 1e-12)


def _img_stats(step, cache, tok_ref, q_ref, as_ref, cm_ref, cnt_ref, m):
    tok = tok_ref[...]
    n = jnp.sqrt(jnp.sum(tok * tok, axis=1, keepdims=True))
    invn = 1.0 / jnp.clip(n, 1e-12)
    b = tok * invn
    s = jax.lax.dot_general(b, cache, (((1,), (1,)), ((), ())))
    rv = jnp.max(s, axis=1, keepdims=True)
    jidx = jax.lax.broadcasted_iota(jnp.int32, s.shape, 1)
    amin = jnp.min(jnp.where(s == rv, jidx, m), axis=1, keepdims=True)
    q_ref[...] = b * jnp.exp(rv)
    as_ref[...] = amin
    pcm = jnp.max(s, axis=0, keepdims=True)
    pc = jnp.sum((jidx == amin).astype(jnp.float32), axis=0, keepdims=True)

    @pl.when(step == 0)
    def _():
        cm_ref[...] = pcm
        cnt_ref[...] = pc

    @pl.when(step != 0)
    def _():
        cm_ref[...] = jnp.maximum(cm_ref[...], pcm)
        cnt_ref[...] = cnt_ref[...] + pc


def _stats_kernel(i4_ref, i8_ref, i12_ref, cache_ref,
                  sc4_ref, as4_ref, cm4_ref, cnt4_ref,
                  sc8_ref, as8_ref, cm8_ref, cnt8_ref,
                  sc12_ref, as12_ref, cm12_ref, cnt12_ref, *, m):
    step = pl.program_id(0)
    cache = cache_ref[...]
    _img_stats(step, cache, i4_ref, sc4_ref, as4_ref, cm4_ref, cnt4_ref, m)
    _img_stats(step, cache, i8_ref, sc8_ref, as8_ref, cm8_ref, cnt8_ref, m)
    _img_stats(step, cache, i12_ref, sc12_ref, as12_ref, cm12_ref, cnt12_ref, m)


def _read_kernel(text_ref, cache_ref, w1_ref, w2_ref, tf_ref, loss_ref):
    step = pl.program_id(0)
    cache = cache_ref[...]
    text = text_ref[...]
    base = _rownorm(text)
    s = jax.lax.dot_general(base, cache, (((1,), (1,)), ((), ())))
    p = jnp.exp(s - jnp.max(s, axis=1, keepdims=True))
    p = p / jnp.sum(p, axis=1, keepdims=True)
    fine = jax.lax.dot_general(p, cache, (((1,), (0,)), ((), ())))
    tf = ALPHA * (jax.lax.dot_general(text, w1_ref[...], (((1,), (1,)), ((), ())))
                  + jax.lax.dot_general(fine, w2_ref[...], (((1,), (1,)), ((), ())))) + text
    tf_ref[...] = tf
    ab = jnp.abs(_rownorm(tf) - text)
    pa = jnp.sum(jnp.sum(ab, axis=1, keepdims=True), axis=0, keepdims=True)

    @pl.when(step == 0)
    def _():
        loss_ref[...] = pa

    @pl.when(step != 0)
    def _():
        loss_ref[...] = loss_ref[...] + pa


def _sc_body(q4_ref, q8_ref, q12_ref, as4_ref, as8_ref, as12_ref,
             o4_ref, o8_ref, o12_ref,
             tok_v, as_v, sums_v, *, c, m, nc):
    cid = lax.axis_index("c")
    sid = lax.axis_index("s")
    wid = sid * nc + cid
    lane0 = wid * 16
    lanes = lax.iota(jnp.int32, 16)

    for q_hbm, asn_hbm, out_hbm in ((q4_ref, as4_ref, o4_ref),
                                    (q8_ref, as8_ref, o8_ref),
                                    (q12_ref, as12_ref, o12_ref)):
        pltpu.sync_copy(asn_hbm, as_v)

        @functools.partial(plsc.parallel_loop, 0, m, unroll=8)
        def _(i):
            sums_v[i, :] = jnp.zeros((16,), jnp.float32)

        def chunk_body(ci, _):
            base = ci * TOK_CHUNK
            pltpu.sync_copy(
                q_hbm.at[pl.ds(base, TOK_CHUNK), pl.ds(lane0, 16)], tok_v)

            @functools.partial(plsc.parallel_loop, 0, TOK_CHUNK, unroll=8)
            def _(r):
                iv = jnp.zeros((16,), jnp.int32) + (base + r)
                av = plsc.load_gather(as_v, [iv])
                plsc.addupdate_scatter(sums_v, [av, lanes], tok_v[r, :])
            return 0
        lax.fori_loop(0, c // TOK_CHUNK, chunk_body, 0)

        pltpu.sync_copy(sums_v, out_hbm.at[:, pl.ds(lane0, 16)])


def _update_kernel(cache_ref, s4_ref, c4_ref, m4_ref, s8_ref, c8_ref, m8_ref,
                   s12_ref, c12_ref, m12_ref, out_ref):
    cache = cache_ref[...]
    acc = None
    for s_ref, c_ref, m_ref in ((s4_ref, c4_ref, m4_ref),
                                (s8_ref, c8_ref, m8_ref),
                                (s12_ref, c12_ref, m12_ref)):
        sums = s_ref[...] * jnp.exp(-m_ref[...])
        upd = jnp.where(c_ref[...] > 0.0,
                        MOMENTUM * cache + (1.0 - MOMENTUM) * sums,
                        cache)
        u = _rownorm(upd)
        acc = u if acc is None else acc + u
    out_ref[...] = acc / 3.0


def kernel(text_token, image_token4, image_token8, image_token12, cache, W):
    c, d = text_token.shape
    m = cache.shape[0]
    nblk = c // BQ
    w1 = W[:, :d]
    w2 = W[:, d:]
    f32 = jnp.float32

    tok_spec = pl.BlockSpec((BQ, d), lambda i: (i, 0))
    full2 = lambda shape: pl.BlockSpec(shape, lambda i: (0, 0))
    col_spec = pl.BlockSpec((BQ, 1), lambda i: (i, 0))

    stats_out_shape = tuple(
        x for _ in range(3) for x in (
            jax.ShapeDtypeStruct((c, d), f32),
            jax.ShapeDtypeStruct((c, 1), jnp.int32),
            jax.ShapeDtypeStruct((1, m), f32),
            jax.ShapeDtypeStruct((1, m), f32),
        )
    )
    stats_out_spec = tuple(
        x for _ in range(3) for x in (tok_spec, col_spec, full2((1, m)),
                                      full2((1, m)))
    )
    (q4, as4, cm4, cnt4, q8, as8, cm8, cnt8,
     q12, as12, cm12, cnt12) = pl.pallas_call(
        functools.partial(_stats_kernel, m=m),
        grid=(nblk,),
        in_specs=[tok_spec, tok_spec, tok_spec, full2((m, d))],
        out_specs=stats_out_spec,
        out_shape=stats_out_shape,
    )(image_token4, image_token8, image_token12, cache)

    text_fine, loss_sum = pl.pallas_call(
        _read_kernel,
        grid=(nblk,),
        in_specs=[tok_spec, full2((m, d)), full2((d, d)), full2((d, d))],
        out_specs=(tok_spec, full2((1, 1))),
        out_shape=(jax.ShapeDtypeStruct((c, d), f32),
                   jax.ShapeDtypeStruct((1, 1), f32)),
    )(text_token, cache, w1, w2)

    info = plsc.get_sparse_core_info()
    nc = info.num_cores
    mesh = plsc.VectorSubcoreMesh(core_axis_name="c", subcore_axis_name="s")
    sc_fn = pl.kernel(
        functools.partial(_sc_body, c=c, m=m, nc=nc),
        mesh=mesh,
        compiler_params=pltpu.CompilerParams(use_tc_tiling_on_sc=False,
                                             needs_layout_passes=False),
        out_type=tuple(jax.ShapeDtypeStruct((m, d), f32) for _ in range(3)),
        scratch_types=[
            pltpu.VMEM((TOK_CHUNK, 16), f32),
            pltpu.VMEM((c,), jnp.int32),
            pltpu.VMEM((m, 16), f32),
        ],
    )
    s4, s8, s12 = sc_fn(q4, q8, q12,
                        as4.reshape(c), as8.reshape(c), as12.reshape(c))

    bm = 688 if m % 688 == 0 else m
    row_spec = pl.BlockSpec((bm, d), lambda i: (i, 0))
    cnt_spec = pl.BlockSpec((bm, 1), lambda i: (i, 0))
    updated_cache = pl.pallas_call(
        _update_kernel,
        grid=(m // bm,),
        in_specs=[row_spec] + [x for _ in range(3)
                               for x in (row_spec, cnt_spec, cnt_spec)],
        out_specs=row_spec,
        out_shape=jax.ShapeDtypeStruct((m, d), f32),
    )(cache, s4, cnt4.T, cm4.T, s8, cnt8.T, cm8.T, s12, cnt12.T, cm12.T)

    loss = loss_sum[0, 0] / (c * d)
    return (text_fine, loss, updated_cache)
